# baseline (speedup 1.0000x reference)
import jax
import jax.numpy as jnp
from jax import lax
from jax.experimental import pallas as pl
from jax.experimental.pallas import tpu as pltpu
from jax.experimental.pallas import tpu_sc as plsc

_N_FACTORS = 16
_BATCH = 4096

_INFO = plsc.get_sparse_core_info()
_NC = _INFO.num_cores
_NS = _INFO.num_subcores
_NW = _NC * _NS
_G = 16
_L = 16

_N_CHUNKS = 1
_CHUNK_ROWS = _BATCH // _N_CHUNKS


def _iota16():
    return lax.iota(jnp.int32, _L)


def _splat(c):
    return jnp.full((_L,), c, jnp.int32)


def _extract(vec, kvec, t):
    return jnp.sum(vec * jnp.where(kvec == t, 1, 0))


def _gather_vb_body(items_hbm, it_hbm, ib_hbm, v_out, b_out,
                    iidx_v, vblocks, bblocks, vcols, bvals, sem_v, sem_b):
    bpw = _BATCH // _NW
    wid = lax.axis_index("s") * _NC + lax.axis_index("c")
    base = wid * bpw
    pltpu.sync_copy(items_hbm.at[pl.ds(base, bpw)], iidx_v)
    kvec = _iota16()
    n_groups = bpw // _G

    def fire(g):
        sl = pl.ds(g * _G, _G)
        ai_vec = lax.shift_left(lax.shift_right_logical(iidx_v[sl], 7), 7)
        buf = g % 2
        handles = []
        for t in range(_G):
            ai = pl.multiple_of(_extract(ai_vec, kvec, t), 128)
            handles.append(pltpu.async_copy(
                it_hbm.at[:, pl.ds(ai, 128)], vblocks.at[buf, t], sem_v))
            handles.append(pltpu.async_copy(
                ib_hbm.at[pl.ds(ai, 128)], bblocks.at[buf, t], sem_b))
        return handles

    def consume(g, handles):
        for h in handles:
            h.wait()
        sl = pl.ds(g * _G, _G)
        buf = g % 2
        ilane = lax.bitwise_and(iidx_v[sl], 127)
        outcol = kvec + g * _G
        for r in range(_N_FACTORS):
            vv = plsc.load_gather(vblocks.at[buf], [kvec, _splat(r), ilane])
            plsc.store_scatter(vcols, [_splat(r), outcol], vv)
        bvals[sl] = plsc.load_gather(bblocks.at[buf], [kvec, ilane])

    prev = fire(0)
    for g in range(1, n_groups):
        cur = fire(g)
        consume(g - 1, prev)
        prev = cur
    consume(n_groups - 1, prev)

    pltpu.sync_copy(vcols, v_out.at[:, pl.ds(base, bpw)])
    pltpu.sync_copy(bvals, b_out.at[pl.ds(base, bpw)])


def _gather_u_body(users_hbm, ut_hbm, u_out,
                   uidx_v, ublocks, ucols, sem_u):
    bpw = _CHUNK_ROWS // _NW
    wid = lax.axis_index("s") * _NC + lax.axis_index("c")
    base = wid * bpw
    pltpu.sync_copy(users_hbm.at[pl.ds(base, bpw)], uidx_v)
    kvec = _iota16()
    n_groups = bpw // _G

    def fire(g):
        sl = pl.ds(g * _G, _G)
        au_vec = lax.shift_left(lax.shift_right_logical(uidx_v[sl], 7), 7)
        buf = g % 2
        handles = []
        for t in range(_G):
            au = pl.multiple_of(_extract(au_vec, kvec, t), 128)
            handles.append(pltpu.async_copy(
                ut_hbm.at[:, pl.ds(au, 128)], ublocks.at[buf, t], sem_u))
        return handles

    def consume(g, handles):
        for h in handles:
            h.wait()
        sl = pl.ds(g * _G, _G)
        buf = g % 2
        ulane = lax.bitwise_and(uidx_v[sl], 127)
        outrow = kvec + g * _G
        for r in range(_N_FACTORS):
            uv = plsc.load_gather(ublocks.at[buf], [kvec, _splat(r), ulane])
            plsc.store_scatter(ucols, [outrow, _splat(r)], uv)

    prev = fire(0)
    for g in range(1, n_groups):
        cur = fire(g)
        consume(g - 1, prev)
        prev = cur
    consume(n_groups - 1, prev)

    pltpu.sync_copy(ucols, u_out.at[pl.ds(base, bpw), :])


_MESH = plsc.VectorSubcoreMesh(core_axis_name="c", subcore_axis_name="s")
_SC_PARAMS = pltpu.CompilerParams(needs_layout_passes=False)

_gather_vb = pl.kernel(
    _gather_vb_body,
    mesh=_MESH,
    out_type=[
        jax.ShapeDtypeStruct((_N_FACTORS, _BATCH), jnp.float32),
        jax.ShapeDtypeStruct((_BATCH,), jnp.float32),
    ],
    scratch_types=[
        pltpu.VMEM((_BATCH // _NW,), jnp.int32),
        pltpu.VMEM((2, _G, _N_FACTORS, 128), jnp.float32),
        pltpu.VMEM((2, _G, 128), jnp.float32),
        pltpu.VMEM((_N_FACTORS, _BATCH // _NW), jnp.float32),
        pltpu.VMEM((_BATCH // _NW,), jnp.float32),
        pltpu.SemaphoreType.DMA,
        pltpu.SemaphoreType.DMA,
    ],
    compiler_params=_SC_PARAMS,
)

_gather_u = pl.kernel(
    _gather_u_body,
    mesh=_MESH,
    out_type=[
        jax.ShapeDtypeStruct((_CHUNK_ROWS, _N_FACTORS), jnp.float32),
    ],
    scratch_types=[
        pltpu.VMEM((_CHUNK_ROWS // _NW,), jnp.int32),
        pltpu.VMEM((2, _G, _N_FACTORS, 128), jnp.float32),
        pltpu.VMEM((_CHUNK_ROWS // _NW, _N_FACTORS), jnp.float32),
        pltpu.SemaphoreType.DMA,
    ],
    compiler_params=_SC_PARAMS,
)

_BM = 1024


def _mm_first_body(u_ref, vt_ref, b_ref, o_ref):
    o_ref[...] = lax.dot_general(
        u_ref[...], vt_ref[...],
        (((1,), (0,)), ((), ())),
        preferred_element_type=jnp.float32,
    ) + b_ref[...]


def _mm_chunk_body(u_ref, vt_ref, b_ref, prev_ref, o_ref):
    o_ref[...] = lax.dot_general(
        u_ref[...], vt_ref[...],
        (((1,), (0,)), ((), ())),
        preferred_element_type=jnp.float32,
    ) + b_ref[...]


@jax.jit
def kernel(users, items, user_table, item_table, item_bias):
    ut_t = user_table.T
    v_t, b = _gather_vb(items, item_table.T, item_bias.reshape(-1))
    u_chunks = [
        _gather_u(lax.slice(users, (c * _CHUNK_ROWS,),
                            ((c + 1) * _CHUNK_ROWS,)), ut_t)[0]
        for c in range(_N_CHUNKS)
    ]
    b2 = b.reshape(_BATCH, 1)
    bpg = _CHUNK_ROWS // _BM

    def chunk_call(c, prev):
        b_sl = lax.slice(b2, (c * _CHUNK_ROWS, 0), ((c + 1) * _CHUNK_ROWS, 1))
        common_in = [
            pl.BlockSpec((_BM, _N_FACTORS), lambda i: (i, 0)),
            pl.BlockSpec((_N_FACTORS, _BATCH), lambda i: (0, 0)),
            pl.BlockSpec((_BM, 1), lambda i: (i, 0)),
        ]
        out_spec = pl.BlockSpec((_BM, _BATCH),
                                lambda i, c=c: (c * bpg + i, 0))
        if prev is None:
            return pl.pallas_call(
                _mm_first_body,
                grid=(bpg,),
                in_specs=common_in,
                out_specs=out_spec,
                out_shape=jax.ShapeDtypeStruct((_BATCH, _BATCH), jnp.float32),
            )(u_chunks[c], v_t, b_sl)
        return pl.pallas_call(
            _mm_chunk_body,
            grid=(bpg,),
            in_specs=common_in + [pl.BlockSpec(memory_space=pl.ANY)],
            out_specs=out_spec,
            out_shape=jax.ShapeDtypeStruct((_BATCH, _BATCH), jnp.float32),
            input_output_aliases={3: 0},
        )(u_chunks[c], v_t, b_sl, prev)

    out = None
    for c in range(_N_CHUNKS):
        out = chunk_call(c, out)
    return out


# speedup vs baseline: 1.0436x; 1.0436x over previous
import jax
import jax.numpy as jnp
from jax import lax
from jax.experimental import pallas as pl
from jax.experimental.pallas import tpu as pltpu
from jax.experimental.pallas import tpu_sc as plsc

_N_FACTORS = 16
_BATCH = 4096

_INFO = plsc.get_sparse_core_info()
_NC = _INFO.num_cores
_NS = _INFO.num_subcores
_NW = _NC * _NS
_BPW = _BATCH // _NW
_G = 16
_L = 16


def _iota16():
    return lax.iota(jnp.int32, _L)


def _splat(c):
    return jnp.full((_L,), c, jnp.int32)


def _extract(vec, kvec, t):
    return jnp.sum(vec * jnp.where(kvec == t, 1, 0))


def _gather_body(users_hbm, items_hbm, ut_hbm, it_hbm, ib_hbm,
                 u_out, v_out, b_out,
                 uidx_v, iidx_v, blocks, bblocks, ucols, vcols, bvals,
                 sem_t, sem_b):
    wid = lax.axis_index("s") * _NC + lax.axis_index("c")
    base = wid * _BPW
    pltpu.sync_copy(users_hbm.at[pl.ds(base, _BPW)], uidx_v)
    pltpu.sync_copy(items_hbm.at[pl.ds(base, _BPW)], iidx_v)
    kvec = _iota16()
    n_groups = _BPW // _G

    def fire(j):
        g, buf = j // 2, j % 2
        sl = pl.ds(g * _G, _G)
        idx = uidx_v if buf == 0 else iidx_v
        tbl = ut_hbm if buf == 0 else it_hbm
        avec = lax.shift_left(lax.shift_right_logical(idx[sl], 7), 7)
        handles = []
        for t in range(_G):
            a = pl.multiple_of(_extract(avec, kvec, t), 128)
            handles.append(pltpu.async_copy(
                tbl.at[:, pl.ds(a, 128)], blocks.at[buf, t], sem_t))
            if buf == 1:
                handles.append(pltpu.async_copy(
                    ib_hbm.at[pl.ds(a, 128)], bblocks.at[t], sem_b))
        return handles

    def consume(j, handles):
        for h in handles:
            h.wait()
        g, buf = j // 2, j % 2
        sl = pl.ds(g * _G, _G)
        idx = uidx_v if buf == 0 else iidx_v
        lane = lax.bitwise_and(idx[sl], 127)
        pos = kvec + g * _G
        for r in range(_N_FACTORS):
            vals = plsc.load_gather(blocks.at[buf], [kvec, _splat(r), lane])
            if buf == 0:
                plsc.store_scatter(ucols, [pos, _splat(r)], vals)
            else:
                plsc.store_scatter(vcols, [_splat(r), pos], vals)
        if buf == 1:
            bvals[sl] = plsc.load_gather(bblocks, [kvec, lane])

    n_jobs = 2 * n_groups
    prev = fire(0)
    for j in range(1, n_jobs):
        cur = fire(j)
        consume(j - 1, prev)
        prev = cur
    consume(n_jobs - 1, prev)

    pltpu.sync_copy(ucols, u_out.at[pl.ds(base, _BPW), :])
    pltpu.sync_copy(vcols, v_out.at[:, pl.ds(base, _BPW)])
    pltpu.sync_copy(bvals, b_out.at[pl.ds(base, _BPW)])


_gather = pl.kernel(
    _gather_body,
    mesh=plsc.VectorSubcoreMesh(core_axis_name="c", subcore_axis_name="s"),
    out_type=[
        jax.ShapeDtypeStruct((_BATCH, _N_FACTORS), jnp.float32),
        jax.ShapeDtypeStruct((_N_FACTORS, _BATCH), jnp.float32),
        jax.ShapeDtypeStruct((_BATCH,), jnp.float32),
    ],
    scratch_types=[
        pltpu.VMEM((_BPW,), jnp.int32),
        pltpu.VMEM((_BPW,), jnp.int32),
        pltpu.VMEM((2, _G, _N_FACTORS, 128), jnp.float32),
        pltpu.VMEM((_G, 128), jnp.float32),
        pltpu.VMEM((_BPW, _N_FACTORS), jnp.float32),
        pltpu.VMEM((_N_FACTORS, _BPW), jnp.float32),
        pltpu.VMEM((_BPW,), jnp.float32),
        pltpu.SemaphoreType.DMA,
        pltpu.SemaphoreType.DMA,
    ],
    compiler_params=pltpu.CompilerParams(needs_layout_passes=False),
)

_BM = 512


def _mm_body(u_ref, vt_ref, b_ref, o_ref):
    o_ref[...] = lax.dot_general(
        u_ref[...], vt_ref[...],
        (((1,), (0,)), ((), ())),
        preferred_element_type=jnp.float32,
    ) + b_ref[...]


@jax.jit
def kernel(users, items, user_table, item_table, item_bias):
    u, v_t, b = _gather(users, items, user_table.T, item_table.T,
                        item_bias.reshape(-1))
    return pl.pallas_call(
        _mm_body,
        grid=(_BATCH // _BM,),
        in_specs=[
            pl.BlockSpec((_BM, _N_FACTORS), lambda i: (i, 0)),
            pl.BlockSpec((_N_FACTORS, _BATCH), lambda i: (0, 0)),
            pl.BlockSpec((_BM, 1), lambda i: (i, 0)),
        ],
        out_specs=pl.BlockSpec((_BM, _BATCH), lambda i: (i, 0)),
        out_shape=jax.ShapeDtypeStruct((_BATCH, _BATCH), jnp.float32),
    )(u, v_t, b.reshape(_BATCH, 1))
